# baseline (speedup 1.0000x reference)
import jax
import jax.numpy as jnp
from jax.experimental import pallas as pl

_N = 16384
_C = 1000
_R = 512


def _onehot_block(ids_ref, out_ref):
    ids = ids_ref[0, 0, :].astype(jnp.int32)
    col = jax.lax.broadcasted_iota(jnp.int32, (_R, _C), 1)
    out_ref[...] = (ids[:, None] == col).astype(jnp.float32)


def kernel(integers):
    ids = integers.astype(jnp.int32).reshape(_N // _R, 1, _R)
    return pl.pallas_call(
        _onehot_block,
        grid=(_N // _R,),
        in_specs=[pl.BlockSpec((1, 1, _R), lambda i: (i, 0, 0))],
        out_specs=pl.BlockSpec((_R, _C), lambda i: (i, 0)),
        out_shape=jax.ShapeDtypeStruct((_N, _C), jnp.float32),
    )(ids)


# speedup vs baseline: 1.0733x; 1.0733x over previous
import jax
import jax.numpy as jnp
from jax.experimental import pallas as pl

_N = 16384
_C = 1000
_R = 2048


def _onehot_block(ids_ref, out_ref):
    ids = ids_ref[0, 0, :].astype(jnp.int32)
    col = jax.lax.broadcasted_iota(jnp.int32, (_R, _C), 1)
    out_ref[...] = (ids[:, None] == col).astype(jnp.float32)


def kernel(integers):
    ids = integers.astype(jnp.int32).reshape(_N // _R, 1, _R)
    return pl.pallas_call(
        _onehot_block,
        grid=(_N // _R,),
        in_specs=[pl.BlockSpec((1, 1, _R), lambda i: (i, 0, 0))],
        out_specs=pl.BlockSpec((_R, _C), lambda i: (i, 0)),
        out_shape=jax.ShapeDtypeStruct((_N, _C), jnp.float32),
    )(ids)
